# baseline (speedup 1.0000x reference)
import functools

import jax
import jax.numpy as jnp
from jax import lax
from jax.experimental import pallas as pl
from jax.experimental.pallas import tpu as pltpu
from jax.experimental.pallas import tpu_sc as plsc

_NUM_CORES = 2
_NUM_SUBCORES = 16
_NUM_WORKERS = _NUM_CORES * _NUM_SUBCORES


@functools.lru_cache(maxsize=None)
def _make_gather(B, V, D):
    assert B % _NUM_WORKERS == 0
    b_per_w = B // _NUM_WORKERS
    C = 16
    NBUF = 4
    assert b_per_w % C == 0
    n_chunks = b_per_w // C

    mesh = plsc.VectorSubcoreMesh(core_axis_name="c", subcore_axis_name="s")

    @functools.partial(
        pl.kernel,
        mesh=mesh,
        out_type=jax.ShapeDtypeStruct((B, D), jnp.float32),
        scratch_types=[
            pltpu.VMEM((b_per_w,), jnp.int32),
        ] + [pltpu.VMEM((C, D), jnp.float32)] * NBUF
          + [pltpu.SemaphoreType.DMA] * (2 * NBUF),
    )
    def gather_kernel(idx_hbm, table_hbm, out_hbm, idx_v, *rest):
        bufs = rest[:NBUF]
        gsems = rest[NBUF:2 * NBUF]
        wsems = rest[2 * NBUF:]
        wid = lax.axis_index("s") * _NUM_CORES + lax.axis_index("c")
        base = wid * b_per_w
        pltpu.sync_copy(idx_hbm.at[pl.ds(base, b_per_w)], idx_v)
        gcopy = [None] * NBUF
        wcopy = [None] * NBUF
        for j in range(min(NBUF, n_chunks)):
            gcopy[j] = pltpu.async_copy(
                table_hbm.at[idx_v.at[pl.ds(j * C, C)]], bufs[j], gsems[j])
        for j in range(n_chunks):
            b = j % NBUF
            gcopy[b].wait()
            wcopy[b] = pltpu.async_copy(
                bufs[b], out_hbm.at[pl.ds(base + j * C, C)], wsems[b])
            nj = j + NBUF
            if nj < n_chunks:
                wcopy[b].wait()
                gcopy[b] = pltpu.async_copy(
                    table_hbm.at[idx_v.at[pl.ds(nj * C, C)]],
                    bufs[b], gsems[b])
                wcopy[b] = None
        for b in range(NBUF):
            if wcopy[b] is not None:
                wcopy[b].wait()

    return gather_kernel


def kernel(input_ids, table):
    ids = jnp.reshape(input_ids, (-1,)).astype(jnp.int32)
    B = ids.shape[0]
    V, D = table.shape
    out = _make_gather(B, V, D)(ids, table)
    return out.reshape(input_ids.shape[1], input_ids.shape[2], D)


# speedup vs baseline: 1.0121x; 1.0121x over previous
import functools

import jax
import jax.numpy as jnp
from jax import lax
from jax.experimental import pallas as pl
from jax.experimental.pallas import tpu as pltpu
from jax.experimental.pallas import tpu_sc as plsc

_NUM_CORES = 2
_NUM_SUBCORES = 16
_NUM_WORKERS = _NUM_CORES * _NUM_SUBCORES


@functools.lru_cache(maxsize=None)
def _make_gather(R, S, V, D):
    B = R * S
    assert B % _NUM_WORKERS == 0
    b_per_w = B // _NUM_WORKERS
    assert S % b_per_w == 0
    w_per_row = S // b_per_w
    C = 32
    NBUF = 2
    assert b_per_w % (C * NBUF) == 0
    n_chunks = b_per_w // C

    mesh = plsc.VectorSubcoreMesh(core_axis_name="c", subcore_axis_name="s")

    @functools.partial(
        pl.kernel,
        mesh=mesh,
        out_type=jax.ShapeDtypeStruct((R, S, D), jnp.float32),
        scratch_types=[
            pltpu.VMEM((b_per_w,), jnp.int32),
        ] + [pltpu.VMEM((C, D), jnp.float32)] * NBUF
          + [pltpu.SemaphoreType.DMA] * (2 * NBUF),
    )
    def gather_kernel(idx_hbm, table_hbm, out_hbm, idx_v, *rest):
        bufs = rest[:NBUF]
        gsems = rest[NBUF:2 * NBUF]
        wsems = rest[2 * NBUF:]
        wid = lax.axis_index("s") * _NUM_CORES + lax.axis_index("c")
        r = wid // w_per_row
        col0 = (wid % w_per_row) * b_per_w
        pltpu.sync_copy(idx_hbm.at[0, r, pl.ds(col0, b_per_w)], idx_v)
        for b in range(NBUF):
            pltpu.async_copy(
                table_hbm.at[idx_v.at[pl.ds(b * C, C)]], bufs[b], gsems[b])

        @pl.loop(0, n_chunks, step=NBUF)
        def _(j):
            for b in range(NBUF):
                jj = j + b
                pltpu.make_async_copy(
                    table_hbm.at[idx_v.at[pl.ds(0, C)]], bufs[b],
                    gsems[b]).wait()
                wcopy = pltpu.async_copy(
                    bufs[b], out_hbm.at[r, pl.ds(col0 + jj * C, C), :],
                    wsems[b])
                wcopy.wait()
                nj = jj + NBUF

                @pl.when(nj < n_chunks)
                def _():
                    pltpu.async_copy(
                        table_hbm.at[idx_v.at[pl.ds(nj * C, C)]],
                        bufs[b], gsems[b])

    return gather_kernel


def kernel(input_ids, table):
    _, R, S = input_ids.shape
    V, D = table.shape
    return _make_gather(R, S, V, D)(input_ids, table)
